# baseline (speedup 1.0000x reference)
import jax
import jax.numpy as jnp
from jax import lax
from jax.experimental import pallas as pl
from jax.experimental.pallas import tpu as pltpu
from jax.experimental.pallas import tpu_sc as plsc

D = 14336
NC, NS = 2, 16
NW = NC * NS
B = 512
SC_B = 256
TC_B = B - SC_B
BPW = SC_B // NW
CH = 2
NB = 2
NCHUNK = BPW // CH


def _sc_body(idx_hbm, table_hbm, out_hbm, idx_v, buf0, buf1, g0, g1, w0, w1):
    wid = lax.axis_index("s") * NC + lax.axis_index("c")
    base = wid * BPW
    pltpu.sync_copy(idx_hbm.at[wid], idx_v)
    bufs = (buf0, buf1)
    gsems = (g0, g1)
    wsems = (w0, w1)

    def gather(j, b):
        return pltpu.make_async_copy(
            table_hbm.at[idx_v.at[j]], bufs[b], gsems[b])

    def write(j, b):
        return pltpu.make_async_copy(
            bufs[b], out_hbm.at[pl.ds(base + j * CH, CH)], wsems[b])

    for b in range(NB):
        gather(b, b).start()

    def step(t, carry):
        for b in range(NB):
            j = t * NB + b
            gather(j, b).wait()
            write(j, b).start()
            write(j, b).wait()
            gather(j + NB, b).start()
        return carry

    lax.fori_loop(0, NCHUNK // NB - 1, step, 0)
    for b in range(NB):
        j = NCHUNK - NB + b
        gather(j, b).wait()
        write(j, b).start()
    for b in range(NB):
        write(NCHUNK - NB + b, b).wait()


_sc_call = pl.kernel(
    _sc_body,
    out_type=jax.ShapeDtypeStruct((SC_B, D), jnp.float32),
    mesh=plsc.VectorSubcoreMesh(core_axis_name="c", subcore_axis_name="s"),
    scratch_types=(
        [pltpu.VMEM((NCHUNK, CH), jnp.int32)]
        + [pltpu.VMEM((CH, D), jnp.float32)] * NB
        + [pltpu.SemaphoreType.DMA] * (2 * NB)
    ),
)


_NSEM = 8


def _tc_body(idx_ref, table_ref, out_ref, sems):
    def copy(i, k):
        return pltpu.make_async_copy(
            table_ref.at[pl.ds(idx_ref[i], 1)],
            out_ref.at[pl.ds(i, 1)],
            sems.at[k])

    def step(i, carry):
        @pl.when(i >= _NSEM)
        def _():
            copy(i - _NSEM, lax.rem(i - _NSEM, _NSEM)).wait()
        copy(i, lax.rem(i, _NSEM)).start()
        return carry

    lax.fori_loop(0, TC_B, step, 0)

    def drain(i, carry):
        copy(i, lax.rem(i, _NSEM)).wait()
        return carry

    lax.fori_loop(TC_B - _NSEM, TC_B, drain, 0)


def _tc_call(idx, table):
    return pl.pallas_call(
        _tc_body,
        grid_spec=pltpu.PrefetchScalarGridSpec(
            num_scalar_prefetch=1,
            grid=(1,),
            in_specs=[pl.BlockSpec(memory_space=pl.ANY)],
            out_specs=pl.BlockSpec(memory_space=pl.ANY),
            scratch_shapes=[pltpu.SemaphoreType.DMA((_NSEM,))],
        ),
        out_shape=jax.ShapeDtypeStruct((TC_B, D), jnp.float32),
    )(idx, table)


def kernel(prefix, embedding_table):
    bsz, seq = prefix.shape
    flat = prefix.astype(jnp.int32).reshape(B)
    sc_idx = flat[:SC_B].reshape(NW, NCHUNK, CH)
    tc_idx = flat[SC_B:]
    sc_out = _sc_call(sc_idx, embedding_table)
    tc_out = _tc_call(tc_idx, embedding_table)
    out = jnp.concatenate([sc_out, tc_out], axis=0)
    return out.reshape(bsz, seq, D)


# speedup vs baseline: 6.9445x; 6.9445x over previous
import jax
import jax.numpy as jnp
from jax import lax
from jax.experimental import pallas as pl
from jax.experimental.pallas import tpu as pltpu
from jax.experimental.pallas import tpu_sc as plsc

D = 14336
NC, NS = 2, 16
NW = NC * NS
B = 512
SC_B = 256
TC_B = B - SC_B
BPW = SC_B // NW
CH = 2
NB = 2
NCHUNK = BPW // CH


def _sc_body(idx_hbm, table_hbm, out_hbm, idx_v, buf0, buf1, g0, g1, w0, w1):
    wid = lax.axis_index("s") * NC + lax.axis_index("c")
    base = wid * BPW
    pltpu.sync_copy(idx_hbm.at[wid], idx_v)
    bufs = (buf0, buf1)
    gsems = (g0, g1)
    wsems = (w0, w1)

    def gather(j, b):
        return pltpu.make_async_copy(
            table_hbm.at[idx_v.at[j]], bufs[b], gsems[b])

    def write(j, b):
        return pltpu.make_async_copy(
            bufs[b], out_hbm.at[pl.ds(base + j * CH, CH)], wsems[b])

    for b in range(NB):
        gather(b, b).start()

    def step(t, carry):
        for b in range(NB):
            j = t * NB + b
            gather(j, b).wait()
            write(j, b).start()
            write(j, b).wait()
            gather(j + NB, b).start()
        return carry

    lax.fori_loop(0, NCHUNK // NB - 1, step, 0)
    for b in range(NB):
        j = NCHUNK - NB + b
        gather(j, b).wait()
        write(j, b).start()
    for b in range(NB):
        write(NCHUNK - NB + b, b).wait()


_sc_call = pl.kernel(
    _sc_body,
    out_type=jax.ShapeDtypeStruct((SC_B, D), jnp.float32),
    mesh=plsc.VectorSubcoreMesh(core_axis_name="c", subcore_axis_name="s"),
    scratch_types=(
        [pltpu.VMEM((NCHUNK, CH), jnp.int32)]
        + [pltpu.VMEM((CH, D), jnp.float32)] * NB
        + [pltpu.SemaphoreType.DMA] * (2 * NB)
    ),
)


_TCBLK = 8
_V = 128


def _tc_body(idx_ref, table_ref, out_ref):
    i = pl.program_id(0)
    cols = lax.broadcasted_iota(jnp.int32, (1, _V), 1)
    onehot = jnp.concatenate(
        [(cols == idx_ref[i * _TCBLK + k]).astype(jnp.float32)
         for k in range(_TCBLK)], axis=0)
    out_ref[...] = jnp.dot(onehot, table_ref[...],
                           preferred_element_type=jnp.float32)


def _tc_call(idx, table):
    return pl.pallas_call(
        _tc_body,
        grid_spec=pltpu.PrefetchScalarGridSpec(
            num_scalar_prefetch=1,
            grid=(TC_B // _TCBLK,),
            in_specs=[pl.BlockSpec((_V, D), lambda i, idx_ref: (0, 0))],
            out_specs=pl.BlockSpec((_TCBLK, D), lambda i, idx_ref: (i, 0)),
        ),
        out_shape=jax.ShapeDtypeStruct((TC_B, D), jnp.float32),
    )(idx, table)


def kernel(prefix, embedding_table):
    bsz, seq = prefix.shape
    flat = prefix.astype(jnp.int32).reshape(B)
    sc_idx = flat[:SC_B].reshape(NW, NCHUNK, CH)
    tc_idx = flat[SC_B:]
    sc_out = _sc_call(sc_idx, embedding_table)
    tc_out = _tc_call(tc_idx, embedding_table)
    out = jnp.concatenate([sc_out, tc_out], axis=0)
    return out.reshape(bsz, seq, D)


# speedup vs baseline: 11.5116x; 1.6577x over previous
import jax
import jax.numpy as jnp
from jax import lax
from jax.experimental import pallas as pl
from jax.experimental.pallas import tpu as pltpu
from jax.experimental.pallas import tpu_sc as plsc

D = 14336
NC, NS = 2, 16
NW = NC * NS
B = 512
BPW = B // NW
CH = 2
NB = 2
NCHUNK = BPW // CH


def _body(idx_hbm, table_hbm, out_hbm, idx_v, buf0, buf1, g0, g1, w0, w1):
    wid = lax.axis_index("s") * NC + lax.axis_index("c")
    base = wid * BPW
    pltpu.sync_copy(idx_hbm.at[wid], idx_v)
    bufs = (buf0, buf1)
    gsems = (g0, g1)
    wsems = (w0, w1)

    def gather(j, b):
        return pltpu.make_async_copy(
            table_hbm.at[idx_v.at[j]], bufs[b], gsems[b])

    def write(j, b):
        return pltpu.make_async_copy(
            bufs[b], out_hbm.at[pl.ds(base + j * CH, CH)], wsems[b])

    for b in range(NB):
        gather(b, b).start()
    def step(t, carry):
        for b in range(NB):
            j = t * NB + b
            gather(j, b).wait()
            write(j, b).start()
            write(j, b).wait()
            gather(j + NB, b).start()
        return carry
    lax.fori_loop(0, NCHUNK // NB - 1, step, 0)
    for b in range(NB):
        j = NCHUNK - NB + b
        gather(j, b).wait()
        write(j, b).start()
    for b in range(NB):
        write(NCHUNK - NB + b, b).wait()


_gather_call = pl.kernel(
    _body,
    out_type=jax.ShapeDtypeStruct((B, D), jnp.float32),
    mesh=plsc.VectorSubcoreMesh(core_axis_name="c", subcore_axis_name="s"),
    scratch_types=(
        [pltpu.VMEM((NCHUNK, CH), jnp.int32)]
        + [pltpu.VMEM((CH, D), jnp.float32)] * NB
        + [pltpu.SemaphoreType.DMA] * (2 * NB)
    ),
)


def kernel(prefix, embedding_table):
    bsz, seq = prefix.shape
    idx = prefix.astype(jnp.int32).reshape(NW, NCHUNK, CH)
    out = _gather_call(idx, embedding_table)
    return out.reshape(bsz, seq, D)
